# baseline (speedup 1.0000x reference)
import functools

import jax
import jax.numpy as jnp
from jax import lax
from jax.experimental import pallas as pl
from jax.experimental.pallas import tpu as pltpu
from jax.experimental.pallas import tpu_sc as plsc

_NC = 2
_NS = 16
_NW = _NC * _NS
_L = 16


def _dot_scores(x, src, dst):
    n_nodes, d_words = x.shape
    n_edges = src.shape[0]
    epw = n_edges // _NW
    chunk = 80
    n_chunks = epw // chunk
    phase_chunks = 25
    phase_edges = phase_chunks * chunk
    n_phases = n_chunks // phase_chunks
    assert epw * _NW == n_edges and n_chunks * chunk == epw
    assert n_phases * phase_chunks == n_chunks
    n_grp = chunk // _L

    mesh = plsc.VectorSubcoreMesh(core_axis_name="c", subcore_axis_name="s")

    @functools.partial(
        pl.kernel,
        mesh=mesh,
        compiler_params=pltpu.CompilerParams(
            needs_layout_passes=False, use_tc_tiling_on_sc=False),
        out_type=jax.ShapeDtypeStruct((n_edges,), jnp.float32),
        scratch_types=[
            pltpu.VMEM((phase_edges,), jnp.int32),
            pltpu.VMEM((phase_edges,), jnp.int32),
            pltpu.VMEM((2, chunk, d_words), jnp.int32),
            pltpu.VMEM((2, chunk, d_words), jnp.int32),
            pltpu.VMEM((2, chunk), jnp.float32),
            pltpu.VMEM((_L, 17), jnp.float32),
            pltpu.VMEM_SHARED((n_nodes, d_words), jnp.int32),
            pltpu.SemaphoreType.DMA((2,)),
            pltpu.SemaphoreType.DMA((2,)),
            pltpu.SemaphoreType.DMA((2,)),
        ],
    )
    def k(x_hbm, src_hbm, dst_hbm, out_hbm,
          idx_s, idx_d, rows_s, rows_d, out_v, pbuf, x_sh,
          sem_s, sem_d, sem_o):
        wid = lax.axis_index("s") * _NC + lax.axis_index("c")
        base = wid * epw
        sid = lax.axis_index("s")
        rpt = (n_nodes // _NS) // 8 * 8
        rem = n_nodes - rpt * _NS
        pltpu.sync_copy(x_hbm.at[pl.ds(sid * rpt, rpt)],
                        x_sh.at[pl.ds(sid * rpt, rpt)])
        if rem:
            @pl.when(sid == 0)
            def _():
                pltpu.sync_copy(x_hbm.at[pl.ds(rpt * _NS, rem)],
                                x_sh.at[pl.ds(rpt * _NS, rem)])
        plsc.subcore_barrier()

        lane = jnp.arange(_L, dtype=jnp.int32)

        def load_phase(p):
            pltpu.sync_copy(src_hbm.at[pl.ds(base + p * phase_edges,
                                             phase_edges)], idx_s)
            pltpu.sync_copy(dst_hbm.at[pl.ds(base + p * phase_edges,
                                             phase_edges)], idx_d)

        def fire(j, buf):
            poff = (j % phase_chunks) * chunk
            pltpu.async_copy(x_sh.at[idx_s.at[pl.ds(poff, chunk)]],
                             rows_s.at[buf], sem_s.at[buf])
            pltpu.async_copy(x_sh.at[idx_d.at[pl.ds(poff, chunk)]],
                             rows_d.at[buf], sem_d.at[buf])

        def wait_rows(buf):
            pltpu.make_async_copy(x_sh.at[pl.ds(0, chunk)], rows_s.at[buf],
                                  sem_s.at[buf]).wait()
            pltpu.make_async_copy(x_sh.at[pl.ds(0, chunk)], rows_d.at[buf],
                                  sem_d.at[buf]).wait()

        def drain_out(j, buf):
            pltpu.make_async_copy(
                out_v.at[buf],
                out_hbm.at[pl.ds(base + j * chunk, chunk)],
                sem_o.at[buf]).wait()

        load_phase(0)
        fire(0, 0)

        def chunk_body(j, carry):
            buf = lax.rem(j, 2)
            nxt = 1 - buf

            wait_rows(buf)

            @pl.when(j + 1 < n_chunks)
            def _():
                @pl.when(lax.rem(j + 1, phase_chunks) == 0)
                def _():
                    load_phase((j + 1) // phase_chunks)
                fire(j + 1, nxt)

            @pl.when(j >= 2)
            def _():
                drain_out(j - 2, buf)

            def gbody(g, carry2):
                for jj in range(_L):
                    e = g * _L + jj
                    p = jnp.zeros((_L,), jnp.float32)
                    for kk in range(d_words // _L):
                        sa = plsc.bitcast(
                            rows_s[buf, e, pl.ds(kk * _L, _L)], jnp.bfloat16)
                        sb = plsc.bitcast(
                            rows_d[buf, e, pl.ds(kk * _L, _L)], jnp.bfloat16)
                        a1, a2 = plsc.unpack(
                            sa, format=plsc.PackFormat.INTERLEAVED)
                        b1, b2 = plsc.unpack(
                            sb, format=plsc.PackFormat.INTERLEAVED)
                        p = p + a1 * b1
                        p = p + a2 * b2
                    pbuf[jj, pl.ds(0, _L)] = p
                acc = plsc.load_gather(pbuf, [lane, jnp.zeros((_L,), jnp.int32)])
                for l in range(1, _L):
                    acc = acc + plsc.load_gather(
                        pbuf, [lane, jnp.full((_L,), l, jnp.int32)])
                out_v[buf, pl.ds(g * _L, _L)] = acc
                return carry2

            lax.fori_loop(0, n_grp, gbody, 0)
            pltpu.async_copy(out_v.at[buf],
                             out_hbm.at[pl.ds(base + j * chunk, chunk)],
                             sem_o.at[buf])
            return carry

        lax.fori_loop(0, n_chunks, chunk_body, 0)
        drain_out(n_chunks - 2, lax.rem(n_chunks - 2, 2))
        drain_out(n_chunks - 1, lax.rem(n_chunks - 1, 2))

    return k(x, src, dst)


def kernel(x, edge_index):
    src = edge_index[0].astype(jnp.int32)
    dst = edge_index[1].astype(jnp.int32)
    n_nodes, d_feat = x.shape
    xp = lax.bitcast_convert_type(
        x.astype(jnp.bfloat16).reshape(n_nodes, d_feat // 2, 2), jnp.int32)
    return _dot_scores(xp, src, dst)


# speedup vs baseline: 1.0999x; 1.0999x over previous
import functools

import jax
import jax.numpy as jnp
from jax import lax
from jax.experimental import pallas as pl
from jax.experimental.pallas import tpu as pltpu
from jax.experimental.pallas import tpu_sc as plsc

_NC = 2
_NS = 16
_NW = _NC * _NS
_L = 16


def _dot_scores(x, src, dst):
    n_nodes, d_words = x.shape
    n_edges = src.shape[0]
    epw = n_edges // _NW
    chunk = 80
    n_chunks = epw // chunk
    phase_chunks = 25
    phase_edges = phase_chunks * chunk
    n_phases = n_chunks // phase_chunks
    assert epw * _NW == n_edges and n_chunks * chunk == epw
    assert n_phases * phase_chunks == n_chunks
    n_grp = chunk // _L

    mesh = plsc.VectorSubcoreMesh(core_axis_name="c", subcore_axis_name="s")

    @functools.partial(
        pl.kernel,
        mesh=mesh,
        compiler_params=pltpu.CompilerParams(
            needs_layout_passes=False, use_tc_tiling_on_sc=False),
        out_type=jax.ShapeDtypeStruct((n_edges,), jnp.float32),
        scratch_types=[
            pltpu.VMEM((phase_edges,), jnp.int32),
            pltpu.VMEM((phase_edges,), jnp.int32),
            pltpu.VMEM((2, chunk, d_words), jnp.int32),
            pltpu.VMEM((2, chunk, d_words), jnp.int32),
            pltpu.VMEM((2, chunk), jnp.float32),
            pltpu.VMEM((_L, 17), jnp.float32),
            pltpu.VMEM_SHARED((n_nodes, d_words), jnp.int32),
            pltpu.SemaphoreType.DMA((2,)),
            pltpu.SemaphoreType.DMA((2,)),
            pltpu.SemaphoreType.DMA((2,)),
        ],
    )
    def k(x_hbm, src_hbm, dst_hbm, out_hbm,
          idx_s, idx_d, rows_s, rows_d, out_v, pbuf, x_sh,
          sem_s, sem_d, sem_o):
        wid = lax.axis_index("s") * _NC + lax.axis_index("c")
        base = wid * epw
        sid = lax.axis_index("s")
        rpt = (n_nodes // _NS) // 8 * 8
        rem = n_nodes - rpt * _NS
        pltpu.sync_copy(x_hbm.at[pl.ds(sid * rpt, rpt)],
                        x_sh.at[pl.ds(sid * rpt, rpt)])
        if rem:
            @pl.when(sid == 0)
            def _():
                pltpu.sync_copy(x_hbm.at[pl.ds(rpt * _NS, rem)],
                                x_sh.at[pl.ds(rpt * _NS, rem)])
        plsc.subcore_barrier()

        lane = jnp.arange(_L, dtype=jnp.int32)

        def load_phase(p):
            pltpu.sync_copy(src_hbm.at[pl.ds(base + p * phase_edges,
                                             phase_edges)], idx_s)
            pltpu.sync_copy(dst_hbm.at[pl.ds(base + p * phase_edges,
                                             phase_edges)], idx_d)

        def fire(j, buf):
            poff = (j % phase_chunks) * chunk
            pltpu.async_copy(x_sh.at[idx_s.at[pl.ds(poff, chunk)]],
                             rows_s.at[buf], sem_s.at[buf])
            pltpu.async_copy(x_sh.at[idx_d.at[pl.ds(poff, chunk)]],
                             rows_d.at[buf], sem_d.at[buf])

        def wait_rows(buf):
            pltpu.make_async_copy(x_sh.at[pl.ds(0, chunk)], rows_s.at[buf],
                                  sem_s.at[buf]).wait()
            pltpu.make_async_copy(x_sh.at[pl.ds(0, chunk)], rows_d.at[buf],
                                  sem_d.at[buf]).wait()

        def drain_out(j, buf):
            pltpu.make_async_copy(
                out_v.at[buf],
                out_hbm.at[pl.ds(base + j * chunk, chunk)],
                sem_o.at[buf]).wait()

        load_phase(0)
        fire(0, 0)

        def chunk_body(j, carry):
            buf = lax.rem(j, 2)
            nxt = 1 - buf

            wait_rows(buf)

            @pl.when(j + 1 < n_chunks)
            def _():
                @pl.when(lax.rem(j + 1, phase_chunks) == 0)
                def _():
                    load_phase((j + 1) // phase_chunks)
                fire(j + 1, nxt)

            @pl.when(j >= 2)
            def _():
                drain_out(j - 2, buf)

            def gbody(g, carry2):
                for jj in range(_L):
                    e = g * _L + jj
                    prods = []
                    for kk in range(d_words // _L):
                        sa = plsc.bitcast(
                            rows_s[buf, e, pl.ds(kk * _L, _L)], jnp.bfloat16)
                        sb = plsc.bitcast(
                            rows_d[buf, e, pl.ds(kk * _L, _L)], jnp.bfloat16)
                        prods.append(sa * sb)
                    while len(prods) > 1:
                        prods = [prods[i] + prods[i + 1]
                                 for i in range(0, len(prods), 2)]
                    u1, u2 = plsc.unpack(
                        prods[0], format=plsc.PackFormat.INTERLEAVED)
                    pbuf[jj, pl.ds(0, _L)] = u1 + u2
                acc = plsc.load_gather(pbuf, [lane, jnp.zeros((_L,), jnp.int32)])
                for l in range(1, _L):
                    acc = acc + plsc.load_gather(
                        pbuf, [lane, jnp.full((_L,), l, jnp.int32)])
                out_v[buf, pl.ds(g * _L, _L)] = acc
                return carry2

            lax.fori_loop(0, n_grp, gbody, 0)
            pltpu.async_copy(out_v.at[buf],
                             out_hbm.at[pl.ds(base + j * chunk, chunk)],
                             sem_o.at[buf])
            return carry

        lax.fori_loop(0, n_chunks, chunk_body, 0)
        drain_out(n_chunks - 2, lax.rem(n_chunks - 2, 2))
        drain_out(n_chunks - 1, lax.rem(n_chunks - 1, 2))

    return k(x, src, dst)


def kernel(x, edge_index):
    src = edge_index[0].astype(jnp.int32)
    dst = edge_index[1].astype(jnp.int32)
    n_nodes, d_feat = x.shape
    xp = lax.bitcast_convert_type(
        x.astype(jnp.bfloat16).reshape(n_nodes, d_feat // 2, 2), jnp.int32)
    return _dot_scores(xp, src, dst)


# speedup vs baseline: 1.1010x; 1.0010x over previous
import functools

import jax
import jax.numpy as jnp
from jax import lax
from jax.experimental import pallas as pl
from jax.experimental.pallas import tpu as pltpu
from jax.experimental.pallas import tpu_sc as plsc

_NC = 2
_NS = 16
_NW = _NC * _NS
_L = 16


def _dot_scores(x, src, dst):
    n_nodes, d_words = x.shape
    n_edges = src.shape[0]
    epw = n_edges // _NW
    chunk = 80
    n_chunks = epw // chunk
    phase_chunks = 25
    phase_edges = phase_chunks * chunk
    n_phases = n_chunks // phase_chunks
    assert epw * _NW == n_edges and n_chunks * chunk == epw
    assert n_phases * phase_chunks == n_chunks
    n_grp = chunk // _L

    mesh = plsc.VectorSubcoreMesh(core_axis_name="c", subcore_axis_name="s")

    @functools.partial(
        pl.kernel,
        mesh=mesh,
        compiler_params=pltpu.CompilerParams(
            needs_layout_passes=False, use_tc_tiling_on_sc=False),
        out_type=jax.ShapeDtypeStruct((n_edges,), jnp.float32),
        scratch_types=[
            pltpu.VMEM((2, phase_edges), jnp.int32),
            pltpu.VMEM((2, phase_edges), jnp.int32),
            pltpu.VMEM((4, chunk, d_words), jnp.int32),
            pltpu.VMEM((4, chunk, d_words), jnp.int32),
            pltpu.VMEM((4, chunk), jnp.float32),
            pltpu.VMEM((_L, 17), jnp.float32),
            pltpu.VMEM_SHARED((n_nodes, d_words), jnp.int32),
            pltpu.SemaphoreType.DMA((4,)),
            pltpu.SemaphoreType.DMA((4,)),
            pltpu.SemaphoreType.DMA((4,)),
        ],
    )
    def k(x_hbm, src_hbm, dst_hbm, out_hbm,
          idx_s, idx_d, rows_s, rows_d, out_v, pbuf, x_sh,
          sem_s, sem_d, sem_o):
        wid = lax.axis_index("s") * _NC + lax.axis_index("c")
        base = wid * epw
        sid = lax.axis_index("s")
        rpt = (n_nodes // _NS) // 8 * 8
        rem = n_nodes - rpt * _NS
        pltpu.sync_copy(x_hbm.at[pl.ds(sid * rpt, rpt)],
                        x_sh.at[pl.ds(sid * rpt, rpt)])
        if rem:
            @pl.when(sid == 0)
            def _():
                pltpu.sync_copy(x_hbm.at[pl.ds(rpt * _NS, rem)],
                                x_sh.at[pl.ds(rpt * _NS, rem)])
        plsc.subcore_barrier()

        lane = jnp.arange(_L, dtype=jnp.int32)

        def load_phase(p):
            ps = lax.rem(p, 2)
            pltpu.sync_copy(src_hbm.at[pl.ds(base + p * phase_edges,
                                             phase_edges)], idx_s.at[ps])
            pltpu.sync_copy(dst_hbm.at[pl.ds(base + p * phase_edges,
                                             phase_edges)], idx_d.at[ps])

        def fire(j, buf):
            ps = lax.rem(j // phase_chunks, 2)
            poff = lax.rem(j, phase_chunks) * chunk
            pltpu.async_copy(x_sh.at[idx_s.at[ps, pl.ds(poff, chunk)]],
                             rows_s.at[buf], sem_s.at[buf])
            pltpu.async_copy(x_sh.at[idx_d.at[ps, pl.ds(poff, chunk)]],
                             rows_d.at[buf], sem_d.at[buf])

        def wait_rows(buf):
            pltpu.make_async_copy(x_sh.at[pl.ds(0, chunk)], rows_s.at[buf],
                                  sem_s.at[buf]).wait()
            pltpu.make_async_copy(x_sh.at[pl.ds(0, chunk)], rows_d.at[buf],
                                  sem_d.at[buf]).wait()

        def drain_out(j, buf):
            pltpu.make_async_copy(
                out_v.at[buf],
                out_hbm.at[pl.ds(base + j * chunk, chunk)],
                sem_o.at[buf]).wait()

        load_phase(0)
        fire(0, 0)
        fire(1, 1)
        fire(2, 2)

        def chunk_body(j, carry):
            buf = lax.rem(j, 4)

            wait_rows(buf)

            @pl.when(j + 3 < n_chunks)
            def _():
                @pl.when(lax.rem(j + 3, phase_chunks) == 0)
                def _():
                    load_phase((j + 3) // phase_chunks)
                fire(j + 3, lax.rem(j + 3, 4))

            @pl.when(j >= 4)
            def _():
                drain_out(j - 4, buf)

            def gbody(g, carry2):
                for jj in range(_L):
                    e = g * _L + jj
                    prods = []
                    for kk in range(d_words // _L):
                        sa = plsc.bitcast(
                            rows_s[buf, e, pl.ds(kk * _L, _L)], jnp.bfloat16)
                        sb = plsc.bitcast(
                            rows_d[buf, e, pl.ds(kk * _L, _L)], jnp.bfloat16)
                        prods.append(sa * sb)
                    while len(prods) > 1:
                        prods = [prods[i] + prods[i + 1]
                                 for i in range(0, len(prods), 2)]
                    u1, u2 = plsc.unpack(
                        prods[0], format=plsc.PackFormat.INTERLEAVED)
                    pbuf[jj, pl.ds(0, _L)] = u1 + u2
                acc = plsc.load_gather(pbuf, [lane, jnp.zeros((_L,), jnp.int32)])
                for l in range(1, _L):
                    acc = acc + plsc.load_gather(
                        pbuf, [lane, jnp.full((_L,), l, jnp.int32)])
                out_v[buf, pl.ds(g * _L, _L)] = acc
                return carry2

            lax.fori_loop(0, n_grp, gbody, 0)
            pltpu.async_copy(out_v.at[buf],
                             out_hbm.at[pl.ds(base + j * chunk, chunk)],
                             sem_o.at[buf])
            return carry

        lax.fori_loop(0, n_chunks, chunk_body, 0)
        for t in range(n_chunks - 4, n_chunks):
            drain_out(t, lax.rem(t, 4))

    return k(x, src, dst)


def kernel(x, edge_index):
    src = edge_index[0].astype(jnp.int32)
    dst = edge_index[1].astype(jnp.int32)
    n_nodes, d_feat = x.shape
    xp = lax.bitcast_convert_type(
        x.astype(jnp.bfloat16).reshape(n_nodes, d_feat // 2, 2), jnp.int32)
    return _dot_scores(xp, src, dst)
